# Initial kernel scaffold; baseline (speedup 1.0000x reference)
import jax
import jax.numpy as jnp
from jax.experimental import pallas as pl


def kernel(input, mempool):
    raise NotImplementedError("write your pallas kernel here")


# speedup vs baseline: 6.5527x; 6.5527x over previous
import jax
import jax.numpy as jnp
from jax.experimental import pallas as pl

_DIM = 96
_NITEM = 1024
_K = 10
_CBLK = 640


def _mem_kernel(x_ref, mem_ref, mt_ref, out_ref, loss_ref):
    b = pl.program_id(0)
    j = pl.program_id(1)
    mem = mem_ref[...]
    mt = mt_ref[...]
    x = x_ref[0]

    logits = jnp.dot(mem, x, preferred_element_type=jnp.float32)
    m0 = jnp.max(logits, axis=0, keepdims=True)
    e = jnp.exp(logits - m0)
    inv_z = 1.0 / jnp.sum(e, axis=0, keepdims=True)

    iota = jax.lax.broadcasted_iota(jnp.int32, (_NITEM, _CBLK), 0)
    work = e
    acc = jnp.zeros_like(e)
    dsum = jnp.zeros_like(inv_z)
    for _ in range(_K):
        cur = jnp.max(work, axis=0, keepdims=True)
        amin = jnp.min(jnp.where(work == cur, iota, _NITEM),
                       axis=0, keepdims=True)
        sel = iota == amin
        w = jnp.exp(cur * inv_z)
        acc = acc + jnp.where(sel, w, 0.0)
        dsum = dsum + w
        work = jnp.where(sel, -1.0, work)
    att = acc * (1.0 / dsum)

    out_ref[0] = jnp.dot(mt, att, preferred_element_type=jnp.float32)

    @pl.when(jnp.logical_and(b == 0, j == 0))
    def _loss():
        cos = jnp.dot(mem, mt, preferred_element_type=jnp.float32) * 0.5
        ii = jax.lax.broadcasted_iota(jnp.int32, (_NITEM, _NITEM), 0)
        jj = jax.lax.broadcasted_iota(jnp.int32, (_NITEM, _NITEM), 1)
        loss_ref[...] = jnp.sum(jnp.where(ii == jj, 0.0, jnp.abs(cos)),
                                axis=(0, 1), keepdims=True)


def kernel(input, mempool):
    B, CH, H, W = input.shape
    hw = H * W
    hwp = ((hw + _CBLK - 1) // _CBLK) * _CBLK
    x = input.reshape(B, CH, hw)
    if hwp != hw:
        x = jnp.pad(x, ((0, 0), (0, 0), (0, hwp - hw)))
    mt = mempool.T
    out, loss = pl.pallas_call(
        _mem_kernel,
        grid=(B, hwp // _CBLK),
        in_specs=[
            pl.BlockSpec((1, CH, _CBLK), lambda b, j: (b, 0, j)),
            pl.BlockSpec((_NITEM, _DIM), lambda b, j: (0, 0)),
            pl.BlockSpec((_DIM, _NITEM), lambda b, j: (0, 0)),
        ],
        out_specs=[
            pl.BlockSpec((1, CH, _CBLK), lambda b, j: (b, 0, j)),
            pl.BlockSpec((1, 1), lambda b, j: (0, 0)),
        ],
        out_shape=[
            jax.ShapeDtypeStruct((B, CH, hwp), jnp.float32),
            jax.ShapeDtypeStruct((1, 1), jnp.float32),
        ],
    )(x, mempool, mt)
    out = out[:, :, :hw] if hwp != hw else out
    return out.reshape(B, CH, H, W), loss[0, 0] / (_NITEM * _NITEM)


# speedup vs baseline: 11.3848x; 1.7374x over previous
import jax
import jax.numpy as jnp
from jax.experimental import pallas as pl

_DIM = 96
_NITEM = 1024
_K = 10
_CBLK = 640
_NEG = -1e30


def _mem_kernel(x_ref, mem_ref, mt_ref, out_ref, loss_ref):
    b = pl.program_id(0)
    j = pl.program_id(1)
    mem = mem_ref[...]
    mt = mt_ref[...]
    x = x_ref[0]

    logits = jnp.dot(mem, x, preferred_element_type=jnp.float32)
    m0 = jnp.max(logits, axis=0, keepdims=True)
    e = jnp.exp(logits - m0)
    inv_z = 1.0 / jnp.sum(e, axis=0, keepdims=True)

    work = logits
    for _ in range(_K):
        cur = jnp.max(work, axis=0, keepdims=True)
        work = jnp.where(work == cur, _NEG, work)
    removed = work == _NEG

    att0 = jnp.where(removed, jnp.exp(e * inv_z), 0.0)
    att = att0 * (1.0 / jnp.sum(att0, axis=0, keepdims=True))

    out_ref[0] = jnp.dot(mt, att, preferred_element_type=jnp.float32)

    @pl.when(jnp.logical_and(b == 0, j == 0))
    def _loss():
        cos = jnp.dot(mem, mt, preferred_element_type=jnp.float32) * 0.5
        ii = jax.lax.broadcasted_iota(jnp.int32, (_NITEM, _NITEM), 0)
        jj = jax.lax.broadcasted_iota(jnp.int32, (_NITEM, _NITEM), 1)
        loss_ref[...] = jnp.sum(jnp.where(ii == jj, 0.0, jnp.abs(cos)),
                                axis=(0, 1), keepdims=True)


def kernel(input, mempool):
    B, CH, H, W = input.shape
    hw = H * W
    hwp = ((hw + _CBLK - 1) // _CBLK) * _CBLK
    x = input.reshape(B, CH, hw)
    if hwp != hw:
        x = jnp.pad(x, ((0, 0), (0, 0), (0, hwp - hw)))
    mt = mempool.T
    out, loss = pl.pallas_call(
        _mem_kernel,
        grid=(B, hwp // _CBLK),
        in_specs=[
            pl.BlockSpec((1, CH, _CBLK), lambda b, j: (b, 0, j)),
            pl.BlockSpec((_NITEM, _DIM), lambda b, j: (0, 0)),
            pl.BlockSpec((_DIM, _NITEM), lambda b, j: (0, 0)),
        ],
        out_specs=[
            pl.BlockSpec((1, CH, _CBLK), lambda b, j: (b, 0, j)),
            pl.BlockSpec((1, 1), lambda b, j: (0, 0)),
        ],
        out_shape=[
            jax.ShapeDtypeStruct((B, CH, hwp), jnp.float32),
            jax.ShapeDtypeStruct((1, 1), jnp.float32),
        ],
    )(x, mempool, mt)
    out = out[:, :, :hw] if hwp != hw else out
    return out.reshape(B, CH, H, W), loss[0, 0] / (_NITEM * _NITEM)
